# Initial kernel scaffold; baseline (speedup 1.0000x reference)
import jax
import jax.numpy as jnp
from jax.experimental import pallas as pl


def kernel(x, edge_index, W1, b1, W2, b2, fc1_W, fc1_b, fc2_W, fc2_b):
    raise NotImplementedError("write your pallas kernel here")


# speedup vs baseline: 4.0689x; 4.0689x over previous
import functools

import jax
import jax.numpy as jnp
from jax import lax
from jax.experimental import pallas as pl
from jax.experimental.pallas import tpu as pltpu
from jax.experimental.pallas import tpu_sc as plsc

N, E, D = 10000, 320000, 128
HD = D // 2
NC, NS = 2, 16
ROWS = N // NS
EPT = E // NS
EPTD = E // (NC * NS)
CH = 128
KHOP = 3

def _sc_mesh():
    return plsc.VectorSubcoreMesh(
        core_axis_name="c", subcore_axis_name="s",
        num_cores=NC, num_subcores=NS)


@functools.cache
def _build_deg_kernel():
    return functools.partial(
        pl.kernel,
        out_type=jax.ShapeDtypeStruct((NC * N, 16), jnp.float32),
        mesh=_sc_mesh(),
        compiler_params=pltpu.CompilerParams(use_tc_tiling_on_sc=False),
        scratch_types=[
            pltpu.VMEM((CH,), jnp.int32),
            pltpu.VMEM((16,), jnp.int32),
            pltpu.VMEM((CH, 16), jnp.float32),
            pltpu.VMEM_SHARED((N, 16), jnp.float32),
        ],
    )(_deg_body)


def _deg_body(dst_hbm, zeros_hbm, ones_hbm, out_hbm, idx_v, idxt_v, ones_v,
              acc_sh):
    cid = lax.axis_index("c")
    sid = lax.axis_index("s")
    row0 = sid * ROWS
    pltpu.sync_copy(zeros_hbm, acc_sh.at[pl.ds(row0, ROWS)])
    pltpu.sync_copy(ones_hbm, ones_v)
    plsc.subcore_barrier()

    base = (cid * NS + sid) * EPTD
    nfull = EPTD // CH

    def chunk(i, carry):
        off = base + i * CH
        pltpu.sync_copy(dst_hbm.at[pl.ds(off, CH)], idx_v)
        pltpu.sync_copy(ones_v, acc_sh.at[idx_v], add=True)
        return carry

    lax.fori_loop(0, nfull, chunk, 0)
    tl = EPTD - nfull * CH
    if tl:
        toff = base + nfull * CH
        pltpu.sync_copy(dst_hbm.at[pl.ds(toff, tl)], idxt_v)
        pltpu.sync_copy(ones_v.at[pl.ds(0, tl)], acc_sh.at[idxt_v], add=True)

    plsc.subcore_barrier()
    pltpu.sync_copy(acc_sh.at[pl.ds(row0, ROWS)],
                    out_hbm.at[pl.ds(cid * N + row0, ROWS)])


@functools.cache
def _build_round_kernel():
    return functools.partial(
        pl.kernel,
        out_type=[jax.ShapeDtypeStruct((N, HD), jnp.float32),
                  jax.ShapeDtypeStruct((N, HD), jnp.float32)],
        mesh=_sc_mesh(),
        compiler_params=pltpu.CompilerParams(use_tc_tiling_on_sc=False),
        scratch_types=[
            pltpu.VMEM((CH,), jnp.int32),
            pltpu.VMEM((CH,), jnp.int32),
            pltpu.VMEM((32,), jnp.int32),
            pltpu.VMEM((32,), jnp.int32),
            pltpu.VMEM((CH, HD), jnp.float32),
            pltpu.VMEM_SHARED((N, HD), jnp.float32),
            pltpu.SemaphoreType.DMA,
        ],
    )(_round_body)


def _round_body(ga_hbm, gb_hbm, src_hbm, dst_hbm, zeros_hbm, oa_hbm, ob_hbm,
                srcv, dstv, srcvt, dstvt, msgv, acc_sh, sem):
    cid = lax.axis_index("c")
    sid = lax.axis_index("s")
    row0 = sid * ROWS
    pltpu.sync_copy(zeros_hbm, acc_sh.at[pl.ds(row0, ROWS)])
    plsc.subcore_barrier()

    base = sid * EPT
    nfull = EPT // CH
    tl = EPT - nfull * CH

    def do_edges(g_hbm):
        def chunk(i, carry):
            off = base + i * CH
            pltpu.sync_copy(src_hbm.at[pl.ds(off, CH)], srcv)
            pltpu.sync_copy(dst_hbm.at[pl.ds(off, CH)], dstv)
            pltpu.async_copy(g_hbm.at[srcv], msgv, sem).wait()
            pltpu.sync_copy(msgv, acc_sh.at[dstv], add=True)
            return carry

        lax.fori_loop(0, nfull, chunk, 0)
        if tl:
            toff = base + nfull * CH
            pltpu.sync_copy(src_hbm.at[pl.ds(toff, tl)], srcvt)
            pltpu.sync_copy(dst_hbm.at[pl.ds(toff, tl)], dstvt)
            pltpu.async_copy(g_hbm.at[srcvt], msgv.at[pl.ds(0, tl)], sem).wait()
            pltpu.sync_copy(msgv.at[pl.ds(0, tl)], acc_sh.at[dstvt], add=True)

    @pl.when(cid == 0)
    def _():
        do_edges(ga_hbm)

    @pl.when(cid == 1)
    def _():
        do_edges(gb_hbm)

    plsc.subcore_barrier()

    @pl.when(cid == 0)
    def _():
        pltpu.sync_copy(acc_sh.at[pl.ds(row0, ROWS)],
                        oa_hbm.at[pl.ds(row0, ROWS)])

    @pl.when(cid == 1)
    def _():
        pltpu.sync_copy(acc_sh.at[pl.ds(row0, ROWS)],
                        ob_hbm.at[pl.ds(row0, ROWS)])


def _deg_call(dst, zeros16, ones16):
    return _build_deg_kernel()(dst, zeros16, ones16)


def _round_call(ga, gb, src, dst, zeros64):
    return _build_round_kernel()(ga, gb, src, dst, zeros64)


BN = 1000


def _prep_body(x_ref, d0_ref, d1_ref, ga_ref, gb_ref, n16_ref):
    deg = d0_ref[:, 0:1] + d1_ref[:, 0:1]
    norm = jnp.where(deg > 0, lax.rsqrt(jnp.maximum(deg, 1e-12)), 0.0)
    g = x_ref[...] * norm
    ga_ref[...] = g[:, :HD]
    gb_ref[...] = g[:, HD:]
    n16_ref[...] = jnp.broadcast_to(norm, (norm.shape[0], 16))


_prep = pl.pallas_call(
    _prep_body,
    grid=(N // BN,),
    in_specs=[pl.BlockSpec((BN, D), lambda i: (i, 0)),
              pl.BlockSpec((BN, 16), lambda i: (i, 0)),
              pl.BlockSpec((BN, 16), lambda i: (i, 0))],
    out_specs=[pl.BlockSpec((BN, HD), lambda i: (i, 0)),
               pl.BlockSpec((BN, HD), lambda i: (i, 0)),
               pl.BlockSpec((BN, 16), lambda i: (i, 0))],
    out_shape=[jax.ShapeDtypeStruct((N, HD), jnp.float32),
               jax.ShapeDtypeStruct((N, HD), jnp.float32),
               jax.ShapeDtypeStruct((N, 16), jnp.float32)],
)


def _scale_body(a_ref, b_ref, n16_ref, oa_ref, ob_ref):
    n2 = n16_ref[:, 0:1] * n16_ref[:, 0:1]
    oa_ref[...] = a_ref[...] * n2
    ob_ref[...] = b_ref[...] * n2


_scale = pl.pallas_call(
    _scale_body,
    grid=(N // BN,),
    in_specs=[pl.BlockSpec((BN, HD), lambda i: (i, 0)),
              pl.BlockSpec((BN, HD), lambda i: (i, 0)),
              pl.BlockSpec((BN, 16), lambda i: (i, 0))],
    out_specs=[pl.BlockSpec((BN, HD), lambda i: (i, 0)),
               pl.BlockSpec((BN, HD), lambda i: (i, 0))],
    out_shape=[jax.ShapeDtypeStruct((N, HD), jnp.float32),
               jax.ShapeDtypeStruct((N, HD), jnp.float32)],
)


def _layer_body(a_ref, b_ref, n16_ref, w_ref, bias_ref, oa_ref, ob_ref):
    nrm = n16_ref[:, 0:1]
    h = jnp.concatenate([a_ref[...], b_ref[...]], axis=1) * nrm
    h = jnp.maximum(h @ w_ref[...] + bias_ref[...], 0.0) * nrm
    oa_ref[...] = h[:, :HD]
    ob_ref[...] = h[:, HD:]


_layer = pl.pallas_call(
    _layer_body,
    grid=(N // BN,),
    in_specs=[pl.BlockSpec((BN, HD), lambda i: (i, 0)),
              pl.BlockSpec((BN, HD), lambda i: (i, 0)),
              pl.BlockSpec((BN, 16), lambda i: (i, 0)),
              pl.BlockSpec((D, D), lambda i: (0, 0)),
              pl.BlockSpec((1, D), lambda i: (0, 0))],
    out_specs=[pl.BlockSpec((BN, HD), lambda i: (i, 0)),
               pl.BlockSpec((BN, HD), lambda i: (i, 0))],
    out_shape=[jax.ShapeDtypeStruct((N, HD), jnp.float32),
               jax.ShapeDtypeStruct((N, HD), jnp.float32)],
)


def _head_body(a_ref, b_ref, n16_ref, w_ref, bias_ref, fc1w_ref, fc1b_ref,
               fc2wt_ref, fc2b_ref, out_ref, acc_ref):
    i = pl.program_id(0)

    @pl.when(i == 0)
    def _():
        acc_ref[...] = jnp.zeros_like(acc_ref)

    nrm = n16_ref[:, 0:1]
    h = jnp.concatenate([a_ref[...], b_ref[...]], axis=1) * nrm
    h = jnp.maximum(h @ w_ref[...] + bias_ref[...], 0.0)
    acc_ref[...] += jnp.sum(h, axis=0, keepdims=True)

    @pl.when(i == pl.num_programs(0) - 1)
    def _():
        hg = acc_ref[...] * (1.0 / N)
        z = jnp.maximum(hg @ fc1w_ref[...] + fc1b_ref[...], 0.0)
        z2 = jnp.sum(z * fc2wt_ref[...], axis=1, keepdims=True) + fc2b_ref[...]
        z2 = jnp.maximum(z2, 0.0)
        out_ref[...] = 1.0 / (1.0 + jnp.exp(-z2))


_head = pl.pallas_call(
    _head_body,
    grid=(N // BN,),
    in_specs=[pl.BlockSpec((BN, HD), lambda i: (i, 0)),
              pl.BlockSpec((BN, HD), lambda i: (i, 0)),
              pl.BlockSpec((BN, 16), lambda i: (i, 0)),
              pl.BlockSpec((D, D), lambda i: (0, 0)),
              pl.BlockSpec((1, D), lambda i: (0, 0)),
              pl.BlockSpec((D, D), lambda i: (0, 0)),
              pl.BlockSpec((1, D), lambda i: (0, 0)),
              pl.BlockSpec((1, D), lambda i: (0, 0)),
              pl.BlockSpec((1, 1), lambda i: (0, 0))],
    out_specs=pl.BlockSpec((1, 1), lambda i: (0, 0)),
    out_shape=jax.ShapeDtypeStruct((1, 1), jnp.float32),
    scratch_shapes=[pltpu.VMEM((1, D), jnp.float32)],
)


def kernel(x, edge_index, W1, b1, W2, b2, fc1_W, fc1_b, fc2_W, fc2_b):
    src = edge_index[0]
    dst = edge_index[1]
    zeros16 = jnp.zeros((ROWS, 16), jnp.float32)
    zeros64 = jnp.zeros((ROWS, HD), jnp.float32)
    ones16 = jnp.ones((CH, 16), jnp.float32)

    degs = _deg_call(dst, zeros16, ones16)
    ga, gb, n16 = _prep(x, degs[:N], degs[N:])

    b1r = b1.reshape(1, D)
    b2r = b2.reshape(1, D)
    fc1br = fc1_b.reshape(1, D)
    fc2wt = fc2_W.reshape(1, D)
    fc2br = fc2_b.reshape(1, 1)

    for layer in range(2):
        for _ in range(KHOP - 1):
            aa, ab = _round_call(ga, gb, src, dst, zeros64)
            ga, gb = _scale(aa, ab, n16)
        aa, ab = _round_call(ga, gb, src, dst, zeros64)
        if layer == 0:
            ga, gb = _layer(aa, ab, n16, W1, b1r)
        else:
            out = _head(aa, ab, n16, W2, b2r, fc1_W, fc1br, fc2wt, fc2br)
    return out
